# baseline (speedup 1.0000x reference)
import jax
import jax.numpy as jnp
from jax.experimental import pallas as pl

_B, _N, _CIN, _D, _Q, _NCLS = 4, 30000, 4, 32, 100, 20
_P = 16
_N2 = _N // _P
_CF = 3 + _CIN
_PK = _P * _CF
_BLKC = 2048
_NBC = -(-_N // _BLKC)

_INV_SQRT_D = 1.0 / float(_D) ** 0.5


def _coarse(cfp_ref, Win_ref, bin_ref, W2_ref, b2_ref, Wsem_ref, q_ref,
            Wcls_ref, logits_ref, sem_ref, qf_ref):
    c2 = cfp_ref[0][:, 0:3]
    f2 = cfp_ref[0][:, 48:48 + _CIN]
    x2 = jnp.maximum(
        c2 @ Win_ref[0:3, :] + f2 @ Win_ref[3:_CF, :] + bin_ref[...], 0.0)
    l2 = jnp.maximum(x2 @ W2_ref[...] + b2_ref[...], 0.0)
    sem_ref[0] = l2 @ Wsem_ref[...]
    scores = jax.lax.dot_general(
        q_ref[...], l2, (((1,), (1,)), ((), ()))) * _INV_SQRT_D
    scores = scores - jnp.max(scores, axis=-1, keepdims=True)
    e = jnp.exp(scores)
    attn = e / jnp.sum(e, axis=-1, keepdims=True)
    qf = attn @ l2
    qf_ref[0] = qf
    logits_ref[0] = qf @ Wcls_ref[...]


def _masks(cfT_ref, WinT_ref, binc_ref, W0T_ref, b0c_ref, qf_ref, masks_ref):
    cfb = cfT_ref[0].astype(jnp.bfloat16)
    xT = jnp.dot(WinT_ref[...], cfb, preferred_element_type=jnp.float32)
    xT = jnp.maximum(xT + binc_ref[...], 0.0).astype(jnp.bfloat16)
    l0T = jnp.dot(W0T_ref[...], xT, preferred_element_type=jnp.float32)
    l0T = jnp.maximum(l0T + b0c_ref[...], 0.0).astype(jnp.bfloat16)
    masks_ref[0] = jax.lax.dot_general(
        l0T, qf_ref[0], (((0,), (1,)), ((), ())),
        preferred_element_type=jnp.float32)


def kernel(coords, feats, W_in, b_in, W_lvl, b_lvl, W_sem, queries, W_cls):
    cfp = jnp.concatenate(
        [coords.reshape(_B, _N2, _P * 3), feats.reshape(_B, _N2, _P * _CIN)],
        axis=-1)
    cfT = jnp.concatenate(
        [coords.transpose(0, 2, 1), feats.transpose(0, 2, 1)],
        axis=1)
    b_in2 = b_in.reshape(1, _D)
    W0, W2 = W_lvl[0], W_lvl[2]
    b0, b2 = b_lvl[0].reshape(1, _D), b_lvl[2].reshape(1, _D)

    full = lambda *shape: pl.BlockSpec(shape, lambda *_: (0,) * len(shape))
    per_b = lambda *shape: pl.BlockSpec(shape, lambda b, *_: (b, 0, 0))

    logits, sem, qf = pl.pallas_call(
        _coarse,
        grid=(_B,),
        in_specs=[
            per_b(1, _N2, _PK),
            full(_CF, _D),
            full(1, _D),
            full(_D, _D),
            full(1, _D),
            full(_D, _NCLS),
            full(_Q, _D),
            full(_D, _NCLS + 1),
        ],
        out_specs=[
            per_b(1, _Q, _NCLS + 1),
            per_b(1, _N2, _NCLS),
            per_b(1, _Q, _D),
        ],
        out_shape=[
            jax.ShapeDtypeStruct((_B, _Q, _NCLS + 1), jnp.float32),
            jax.ShapeDtypeStruct((_B, _N2, _NCLS), jnp.float32),
            jax.ShapeDtypeStruct((_B, _Q, _D), jnp.float32),
        ],
    )(cfp, W_in, b_in2, W2, b2, W_sem, queries, W_cls)

    masks = pl.pallas_call(
        _masks,
        grid=(_B, _NBC),
        in_specs=[
            pl.BlockSpec((1, _CF, _BLKC), lambda b, j: (b, 0, j)),
            full(_D, _CF),
            full(_D, 1),
            full(_D, _D),
            full(_D, 1),
            per_b(1, _Q, _D),
        ],
        out_specs=pl.BlockSpec((1, _BLKC, _Q), lambda b, j: (b, j, 0)),
        out_shape=jax.ShapeDtypeStruct((_B, _N, _Q), jnp.float32),
    )(cfT, W_in.T.astype(jnp.bfloat16), b_in.reshape(_D, 1),
      W0.T.astype(jnp.bfloat16), b0.reshape(_D, 1),
      qf.astype(jnp.bfloat16))

    return (logits, masks, sem)


# speedup vs baseline: 2.4019x; 2.4019x over previous
import jax
import jax.numpy as jnp
from jax.experimental import pallas as pl

_B, _N, _CIN, _D, _Q, _NCLS = 4, 30000, 4, 32, 100, 20
_N2 = _N // 16
_CF = 3 + _CIN
_BLKC = 6144
_NBC = -(-_N // _BLKC)

_INV_SQRT_D = 1.0 / float(_D) ** 0.5


def _coarse(cfs_ref, WinT_ref, binc_ref, W2T_ref, b2c_ref, Wsem_ref, q_ref,
            Wcls_ref, logits_ref, sem_ref, qf_ref):
    x2T = jnp.maximum(
        WinT_ref[...] @ cfs_ref[0] + binc_ref[...], 0.0)
    l2T = jnp.maximum(W2T_ref[...] @ x2T + b2c_ref[...], 0.0)
    sem_ref[0] = jax.lax.dot_general(
        l2T, Wsem_ref[...], (((0,), (0,)), ((), ())))
    scores = (q_ref[...] @ l2T) * _INV_SQRT_D
    scores = scores - jnp.max(scores, axis=-1, keepdims=True)
    e = jnp.exp(scores)
    attn = e / jnp.sum(e, axis=-1, keepdims=True)
    qf = jax.lax.dot_general(
        attn, l2T, (((1,), (1,)), ((), ())))
    qf_ref[0] = qf
    logits_ref[0] = qf @ Wcls_ref[...]


def _masks(cfT_ref, WinT_ref, binc_ref, W0T_ref, b0c_ref, qf_ref, masks_ref):
    cfb = cfT_ref[0].astype(jnp.bfloat16)
    xT = jnp.dot(WinT_ref[...], cfb, preferred_element_type=jnp.float32)
    xT = jnp.maximum(xT + binc_ref[...], 0.0).astype(jnp.bfloat16)
    l0T = jnp.dot(W0T_ref[...], xT, preferred_element_type=jnp.float32)
    l0T = jnp.maximum(l0T + b0c_ref[...], 0.0).astype(jnp.bfloat16)
    masks_ref[0] = jax.lax.dot_general(
        l0T, qf_ref[0], (((0,), (1,)), ((), ())),
        preferred_element_type=jnp.float32)


def kernel(coords, feats, W_in, b_in, W_lvl, b_lvl, W_sem, queries, W_cls):
    cfT = jnp.concatenate(
        [coords.transpose(0, 2, 1), feats.transpose(0, 2, 1)],
        axis=1)
    cfs = cfT[:, :, ::16]

    WinT = W_in.T
    W2T = W_lvl[2].T
    binc = b_in.reshape(_D, 1)
    b2c = b_lvl[2].reshape(_D, 1)

    full = lambda *shape: pl.BlockSpec(shape, lambda *_: (0,) * len(shape))
    per_b = lambda *shape: pl.BlockSpec(shape, lambda b, *_: (b, 0, 0))

    logits, sem, qf = pl.pallas_call(
        _coarse,
        grid=(_B,),
        in_specs=[
            per_b(1, _CF, _N2),
            full(_D, _CF),
            full(_D, 1),
            full(_D, _D),
            full(_D, 1),
            full(_D, _NCLS),
            full(_Q, _D),
            full(_D, _NCLS + 1),
        ],
        out_specs=[
            per_b(1, _Q, _NCLS + 1),
            per_b(1, _N2, _NCLS),
            per_b(1, _Q, _D),
        ],
        out_shape=[
            jax.ShapeDtypeStruct((_B, _Q, _NCLS + 1), jnp.float32),
            jax.ShapeDtypeStruct((_B, _N2, _NCLS), jnp.float32),
            jax.ShapeDtypeStruct((_B, _Q, _D), jnp.float32),
        ],
    )(cfs, WinT, binc, W2T, b2c, W_sem, queries, W_cls)

    masks = pl.pallas_call(
        _masks,
        grid=(_B, _NBC),
        in_specs=[
            pl.BlockSpec((1, _CF, _BLKC), lambda b, j: (b, 0, j)),
            full(_D, _CF),
            full(_D, 1),
            full(_D, _D),
            full(_D, 1),
            per_b(1, _Q, _D),
        ],
        out_specs=pl.BlockSpec((1, _BLKC, _Q), lambda b, j: (b, j, 0)),
        out_shape=jax.ShapeDtypeStruct((_B, _N, _Q), jnp.float32),
    )(cfT, WinT.astype(jnp.bfloat16), binc,
      W_lvl[0].T.astype(jnp.bfloat16), b_lvl[0].reshape(_D, 1),
      qf.astype(jnp.bfloat16))

    return (logits, masks, sem)


# speedup vs baseline: 2.7374x; 1.1397x over previous
import jax
import jax.numpy as jnp
from jax.experimental import pallas as pl

_B, _N, _CIN, _D, _Q, _NCLS = 4, 30000, 4, 32, 100, 20
_N2 = _N // 16
_CF = 3 + _CIN
_BLKC = 6144
_NBC = -(-_N // _BLKC)

_INV_SQRT_D = 1.0 / float(_D) ** 0.5


def _coarse(cfs_ref, WinT_ref, binc_ref, W2T_ref, b2c_ref, Wsem_ref, q_ref,
            Wcls_ref, logits_ref, sem_ref, qf_ref):
    x2T = jnp.maximum(
        WinT_ref[...] @ cfs_ref[0] + binc_ref[...], 0.0)
    l2T = jnp.maximum(W2T_ref[...] @ x2T + b2c_ref[...], 0.0)
    sem_ref[0] = jax.lax.dot_general(
        l2T, Wsem_ref[...], (((0,), (0,)), ((), ())))
    scores = (q_ref[...] @ l2T) * _INV_SQRT_D
    scores = scores - jnp.max(scores, axis=-1, keepdims=True)
    e = jnp.exp(scores)
    attn = e / jnp.sum(e, axis=-1, keepdims=True)
    qf = jax.lax.dot_general(
        attn, l2T, (((1,), (1,)), ((), ())))
    qf_ref[0] = qf
    logits_ref[0] = qf @ Wcls_ref[...]


def _masks(cfT_ref, WinT_ref, binc_ref, W0T_ref, b0c_ref, qf_ref, masks_ref):
    cfb = cfT_ref[0].astype(jnp.bfloat16)
    xT = jnp.dot(WinT_ref[...], cfb, preferred_element_type=jnp.float32)
    xT = jnp.maximum(xT + binc_ref[...], 0.0).astype(jnp.bfloat16)
    l0T = jnp.dot(W0T_ref[...], xT, preferred_element_type=jnp.float32)
    l0T = jnp.maximum(l0T + b0c_ref[...], 0.0).astype(jnp.bfloat16)
    m = jax.lax.dot_general(
        l0T, qf_ref[0], (((0,), (1,)), ((), ())),
        preferred_element_type=jnp.float32)
    masks_ref[0] = m.astype(jnp.bfloat16)


def kernel(coords, feats, W_in, b_in, W_lvl, b_lvl, W_sem, queries, W_cls):
    cfT = jnp.concatenate(
        [coords.transpose(0, 2, 1), feats.transpose(0, 2, 1)],
        axis=1)
    cfs = cfT[:, :, ::16]

    WinT = W_in.T
    W2T = W_lvl[2].T
    binc = b_in.reshape(_D, 1)
    b2c = b_lvl[2].reshape(_D, 1)

    full = lambda *shape: pl.BlockSpec(shape, lambda *_: (0,) * len(shape))
    per_b = lambda *shape: pl.BlockSpec(shape, lambda b, *_: (b, 0, 0))

    logits, sem, qf = pl.pallas_call(
        _coarse,
        grid=(_B,),
        in_specs=[
            per_b(1, _CF, _N2),
            full(_D, _CF),
            full(_D, 1),
            full(_D, _D),
            full(_D, 1),
            full(_D, _NCLS),
            full(_Q, _D),
            full(_D, _NCLS + 1),
        ],
        out_specs=[
            per_b(1, _Q, _NCLS + 1),
            per_b(1, _N2, _NCLS),
            per_b(1, _Q, _D),
        ],
        out_shape=[
            jax.ShapeDtypeStruct((_B, _Q, _NCLS + 1), jnp.float32),
            jax.ShapeDtypeStruct((_B, _N2, _NCLS), jnp.float32),
            jax.ShapeDtypeStruct((_B, _Q, _D), jnp.float32),
        ],
    )(cfs, WinT, binc, W2T, b2c, W_sem, queries, W_cls)

    masks = pl.pallas_call(
        _masks,
        grid=(_B, _NBC),
        in_specs=[
            pl.BlockSpec((1, _CF, _BLKC), lambda b, j: (b, 0, j)),
            full(_D, _CF),
            full(_D, 1),
            full(_D, _D),
            full(_D, 1),
            per_b(1, _Q, _D),
        ],
        out_specs=pl.BlockSpec((1, _BLKC, _Q), lambda b, j: (b, j, 0)),
        out_shape=jax.ShapeDtypeStruct((_B, _N, _Q), jnp.bfloat16),
    )(cfT, WinT.astype(jnp.bfloat16), binc,
      W_lvl[0].T.astype(jnp.bfloat16), b_lvl[0].reshape(_D, 1),
      qf.astype(jnp.bfloat16))

    return (logits, masks.astype(jnp.float32), sem)


# speedup vs baseline: 2.8898x; 1.0557x over previous
import jax
import jax.numpy as jnp
from jax.experimental import pallas as pl
from jax.experimental.pallas import tpu as pltpu

_B, _N, _CIN, _D, _Q, _NCLS = 4, 30000, 4, 32, 100, 20
_N2 = _N // 16
_CF = 3 + _CIN
_BLKC = 7680
_NBC = -(-_N // _BLKC)

_INV_SQRT_D = 1.0 / float(_D) ** 0.5


def _fused(cfs_ref, cfT_ref, WinT_ref, binc_ref, W2T_ref, b2c_ref,
           Wsem_ref, q_ref, Wcls_ref, WinTb_ref, W0Tb_ref, b0c_ref,
           logits_ref, sem_ref, masks_ref, qf_scr):
    j = pl.program_id(1)

    @pl.when(j == 0)
    def _coarse():
        x2T = jnp.maximum(
            WinT_ref[...] @ cfs_ref[0] + binc_ref[...], 0.0)
        l2T = jnp.maximum(W2T_ref[...] @ x2T + b2c_ref[...], 0.0)
        sem_ref[0] = jax.lax.dot_general(
            l2T, Wsem_ref[...], (((0,), (0,)), ((), ())))
        scores = (q_ref[...] @ l2T) * _INV_SQRT_D
        scores = scores - jnp.max(scores, axis=-1, keepdims=True)
        e = jnp.exp(scores)
        attn = e / jnp.sum(e, axis=-1, keepdims=True)
        qf = jax.lax.dot_general(
            attn, l2T, (((1,), (1,)), ((), ())))
        qf_scr[...] = qf.astype(jnp.bfloat16)
        logits_ref[0] = qf @ Wcls_ref[...]

    cfb = cfT_ref[0].astype(jnp.bfloat16)
    xT = jnp.dot(WinTb_ref[...], cfb, preferred_element_type=jnp.float32)
    xT = jnp.maximum(xT + binc_ref[...], 0.0).astype(jnp.bfloat16)
    l0T = jnp.dot(W0Tb_ref[...], xT, preferred_element_type=jnp.float32)
    l0T = jnp.maximum(l0T + b0c_ref[...], 0.0).astype(jnp.bfloat16)
    m = jax.lax.dot_general(
        l0T, qf_scr[...], (((0,), (1,)), ((), ())),
        preferred_element_type=jnp.float32)
    masks_ref[0] = m.astype(jnp.bfloat16)


def kernel(coords, feats, W_in, b_in, W_lvl, b_lvl, W_sem, queries, W_cls):
    cfT = jnp.concatenate(
        [coords.transpose(0, 2, 1), feats.transpose(0, 2, 1)],
        axis=1)
    cfs = cfT[:, :, ::16]

    WinT = W_in.T
    W2T = W_lvl[2].T
    binc = b_in.reshape(_D, 1)
    b2c = b_lvl[2].reshape(_D, 1)

    full = lambda *shape: pl.BlockSpec(shape, lambda *_: (0,) * len(shape))
    per_b = lambda *shape: pl.BlockSpec(shape, lambda b, *_: (b, 0, 0))

    logits, sem, masks = pl.pallas_call(
        _fused,
        grid=(_B, _NBC),
        in_specs=[
            per_b(1, _CF, _N2),
            pl.BlockSpec((1, _CF, _BLKC), lambda b, j: (b, 0, j)),
            full(_D, _CF),
            full(_D, 1),
            full(_D, _D),
            full(_D, 1),
            full(_D, _NCLS),
            full(_Q, _D),
            full(_D, _NCLS + 1),
            full(_D, _CF),
            full(_D, _D),
            full(_D, 1),
        ],
        out_specs=[
            per_b(1, _Q, _NCLS + 1),
            per_b(1, _N2, _NCLS),
            pl.BlockSpec((1, _BLKC, _Q), lambda b, j: (b, j, 0)),
        ],
        out_shape=[
            jax.ShapeDtypeStruct((_B, _Q, _NCLS + 1), jnp.float32),
            jax.ShapeDtypeStruct((_B, _N2, _NCLS), jnp.float32),
            jax.ShapeDtypeStruct((_B, _N, _Q), jnp.bfloat16),
        ],
        scratch_shapes=[pltpu.VMEM((_Q, _D), jnp.bfloat16)],
    )(cfs, cfT, WinT, binc, W2T, b2c, W_sem, queries, W_cls,
      WinT.astype(jnp.bfloat16), W_lvl[0].T.astype(jnp.bfloat16),
      b_lvl[0].reshape(_D, 1))

    return (logits, masks.astype(jnp.float32), sem)


# speedup vs baseline: 2.9975x; 1.0373x over previous
import jax
import jax.numpy as jnp
from jax.experimental import pallas as pl
from jax.experimental.pallas import tpu as pltpu

_B, _N, _CIN, _D, _Q, _NCLS = 4, 30000, 4, 32, 100, 20
_N2 = _N // 16
_CF = 3 + _CIN
_BLKC = 15360
_NBC = -(-_N // _BLKC)

_INV_SQRT_D = 1.0 / float(_D) ** 0.5


def _fused(cfs_ref, cfT_ref, WinT_ref, binc_ref, W2T_ref, b2c_ref,
           Wsem_ref, q_ref, Wcls_ref, WinTb_ref, W0Tb_ref, b0c_ref,
           logits_ref, sem_ref, masks_ref, qf_scr):
    j = pl.program_id(1)

    @pl.when(j == 0)
    def _coarse():
        x2T = jnp.maximum(
            WinT_ref[...] @ cfs_ref[0] + binc_ref[...], 0.0)
        l2T = jnp.maximum(W2T_ref[...] @ x2T + b2c_ref[...], 0.0)
        sem_ref[0] = jax.lax.dot_general(
            l2T, Wsem_ref[...], (((0,), (0,)), ((), ())))
        scores = (q_ref[...] @ l2T) * _INV_SQRT_D
        scores = scores - jnp.max(scores, axis=-1, keepdims=True)
        e = jnp.exp(scores)
        attn = e / jnp.sum(e, axis=-1, keepdims=True)
        qf = jax.lax.dot_general(
            attn, l2T, (((1,), (1,)), ((), ())))
        qf_scr[...] = qf.astype(jnp.bfloat16)
        logits_ref[0] = qf @ Wcls_ref[...]

    cfb = cfT_ref[0].astype(jnp.bfloat16)
    xT = jnp.dot(WinTb_ref[...], cfb, preferred_element_type=jnp.float32)
    xT = jnp.maximum(xT + binc_ref[...], 0.0).astype(jnp.bfloat16)
    l0T = jnp.dot(W0Tb_ref[...], xT, preferred_element_type=jnp.float32)
    l0T = jnp.maximum(l0T + b0c_ref[...], 0.0).astype(jnp.bfloat16)
    m = jax.lax.dot_general(
        l0T, qf_scr[...], (((0,), (1,)), ((), ())),
        preferred_element_type=jnp.float32)
    masks_ref[0] = m.astype(jnp.bfloat16)


def kernel(coords, feats, W_in, b_in, W_lvl, b_lvl, W_sem, queries, W_cls):
    cfT = jnp.concatenate(
        [coords.transpose(0, 2, 1), feats.transpose(0, 2, 1)],
        axis=1)
    cfs = cfT[:, :, ::16]

    WinT = W_in.T
    W2T = W_lvl[2].T
    binc = b_in.reshape(_D, 1)
    b2c = b_lvl[2].reshape(_D, 1)

    full = lambda *shape: pl.BlockSpec(shape, lambda *_: (0,) * len(shape))
    per_b = lambda *shape: pl.BlockSpec(shape, lambda b, *_: (b, 0, 0))

    logits, sem, masks = pl.pallas_call(
        _fused,
        grid=(_B, _NBC),
        in_specs=[
            per_b(1, _CF, _N2),
            pl.BlockSpec((1, _CF, _BLKC), lambda b, j: (b, 0, j)),
            full(_D, _CF),
            full(_D, 1),
            full(_D, _D),
            full(_D, 1),
            full(_D, _NCLS),
            full(_Q, _D),
            full(_D, _NCLS + 1),
            full(_D, _CF),
            full(_D, _D),
            full(_D, 1),
        ],
        out_specs=[
            per_b(1, _Q, _NCLS + 1),
            per_b(1, _N2, _NCLS),
            pl.BlockSpec((1, _BLKC, _Q), lambda b, j: (b, j, 0)),
        ],
        out_shape=[
            jax.ShapeDtypeStruct((_B, _Q, _NCLS + 1), jnp.float32),
            jax.ShapeDtypeStruct((_B, _N2, _NCLS), jnp.float32),
            jax.ShapeDtypeStruct((_B, _N, _Q), jnp.bfloat16),
        ],
        scratch_shapes=[pltpu.VMEM((_Q, _D), jnp.bfloat16)],
    )(cfs, cfT, WinT, binc, W2T, b2c, W_sem, queries, W_cls,
      WinT.astype(jnp.bfloat16), W_lvl[0].T.astype(jnp.bfloat16),
      b_lvl[0].reshape(_D, 1))

    return (logits, masks.astype(jnp.float32), sem)
